# baseline (speedup 1.0000x reference)
import functools

import jax
import jax.numpy as jnp
from jax import lax
from jax.experimental import pallas as pl
from jax.experimental.pallas import tpu as pltpu
from jax.experimental.pallas import tpu_sc as plsc

N = 8192
C = 2048
L = 16
NUM_CORES = 2
NUM_SUBCORES = 16
NW = NUM_CORES * NUM_SUBCORES
ROWS_PER_W = N // NW
GROUPS = ROWS_PER_W // L
BISECT_ITERS = 24


def _body(x_hbm, out_hbm, buf, obuf):
    cid = lax.axis_index("c")
    sid = lax.axis_index("s")
    wid = sid * NUM_CORES + cid
    lanes = lax.iota(jnp.int32, L)

    def col(ref, j):
        return plsc.load_gather(ref, [lanes, lax.broadcast(j, (L,))])

    def do_group(g, carry):
        row0 = wid * ROWS_PER_W + g * L
        pltpu.sync_copy(x_hbm.at[pl.ds(row0, L), :], buf)

        def max_step(j, m):
            return jnp.maximum(m, col(buf, j))

        m = lax.fori_loop(0, C, max_step, jnp.full((L,), -1e30, jnp.float32))

        def bisect(i, lohi):
            lo, hi = lohi
            mid = 0.5 * (lo + hi)

            def acc_step(j, acc):
                return acc + jnp.maximum(col(buf, j) - mid, 0.0)

            s = lax.fori_loop(0, C, acc_step, jnp.zeros((L,), jnp.float32))
            ge = s >= 1.0
            return jnp.where(ge, mid, lo), jnp.where(ge, hi, mid)

        lo, hi = lax.fori_loop(0, BISECT_ITERS, bisect, (m - 1.0, m))

        def ks_step(j, ks):
            k, s = ks
            v = col(buf, j)
            act = v > lo
            return (k + jnp.where(act, 1.0, 0.0), s + jnp.where(act, v, 0.0))

        k, s = lax.fori_loop(
            0, C, ks_step,
            (jnp.zeros((L,), jnp.float32), jnp.zeros((L,), jnp.float32)))
        tau = (s - 1.0) / k

        def out_step(j, _):
            v = jnp.maximum(col(buf, j) - tau, 0.0)
            plsc.store_scatter(obuf, [lanes, lax.broadcast(j, (L,))], v)
            return _

        lax.fori_loop(0, C, out_step, 0)
        pltpu.sync_copy(obuf, out_hbm.at[pl.ds(row0, L), :])
        return carry

    lax.fori_loop(0, GROUPS, do_group, 0)


_sparsemax_sc = functools.partial(
    pl.kernel,
    out_type=jax.ShapeDtypeStruct((N, C), jnp.float32),
    mesh=plsc.VectorSubcoreMesh(
        core_axis_name="c", subcore_axis_name="s",
        num_cores=NUM_CORES, num_subcores=NUM_SUBCORES),
    scratch_types=[
        pltpu.VMEM((L, C), jnp.float32),
        pltpu.VMEM((L, C), jnp.float32),
    ],
    compiler_params=pltpu.CompilerParams(
        use_tc_tiling_on_sc=False, needs_layout_passes=False),
)(_body)


def kernel(input):
    return _sparsemax_sc(input)


# speedup vs baseline: 1.9561x; 1.9561x over previous
import functools

import jax
import jax.numpy as jnp
from jax import lax
from jax.experimental import pallas as pl
from jax.experimental.pallas import tpu as pltpu
from jax.experimental.pallas import tpu_sc as plsc

N = 8192
C = 2048
L = 16
NUM_CORES = 2
NUM_SUBCORES = 16
NW = NUM_CORES * NUM_SUBCORES
ROWS_PER_W = N // NW
GROUPS = ROWS_PER_W // L
BISECT_ITERS = 20
U = 8
GL = L * C


def _body(x_hbm, out_hbm, buf, obuf):
    cid = lax.axis_index("c")
    sid = lax.axis_index("s")
    wid = sid * NUM_CORES + cid
    rowbase = lax.iota(jnp.int32, L) * C

    def col(j):
        return plsc.load_gather(buf, [rowbase + lax.broadcast(j, (L,))])

    def do_group(g, carry):
        base = (wid * GROUPS + g) * GL
        pltpu.sync_copy(x_hbm.at[pl.ds(base, GL)], buf)

        def max_step(i, ms):
            j = i * U
            return tuple(jnp.maximum(ms[u], col(j + u)) for u in range(U))

        ms = lax.fori_loop(0, C // U, max_step,
                           (jnp.full((L,), -1e30, jnp.float32),) * U)
        m = functools.reduce(jnp.maximum, ms)

        def bisect(i, lohi):
            lo, hi = lohi
            mid = 0.5 * (lo + hi)

            def acc_step(i, accs):
                j = i * U
                return tuple(
                    accs[u] + jnp.maximum(col(j + u) - mid, 0.0)
                    for u in range(U))

            accs = lax.fori_loop(0, C // U, acc_step,
                                 (jnp.zeros((L,), jnp.float32),) * U)
            s = functools.reduce(jnp.add, accs)
            ge = s >= 1.0
            return jnp.where(ge, mid, lo), jnp.where(ge, hi, mid)

        lo, hi = lax.fori_loop(0, BISECT_ITERS, bisect, (m - 1.0, m))

        def ks_step(i, ks):
            j = i * U
            ks = list(ks)
            for u in range(U):
                v = col(j + u)
                act = v > lo
                ks[2 * u] = ks[2 * u] + jnp.where(act, 1.0, 0.0)
                ks[2 * u + 1] = ks[2 * u + 1] + jnp.where(act, v, 0.0)
            return tuple(ks)

        ks = lax.fori_loop(0, C // U, ks_step,
                           (jnp.zeros((L,), jnp.float32),) * (2 * U))
        k = functools.reduce(jnp.add, ks[0::2])
        s = functools.reduce(jnp.add, ks[1::2])
        tau = (s - 1.0) / k

        def out_step(i, c):
            j = i * U
            for u in range(U):
                idx = rowbase + lax.broadcast(j + u, (L,))
                v = jnp.maximum(plsc.load_gather(buf, [idx]) - tau, 0.0)
                plsc.store_scatter(obuf, [idx], v)
            return c

        lax.fori_loop(0, C // U, out_step, 0)
        pltpu.sync_copy(obuf, out_hbm.at[pl.ds(base, GL)])
        return carry

    lax.fori_loop(0, GROUPS, do_group, 0)


_sparsemax_sc = functools.partial(
    pl.kernel,
    out_type=jax.ShapeDtypeStruct((N * C,), jnp.float32),
    mesh=plsc.VectorSubcoreMesh(
        core_axis_name="c", subcore_axis_name="s",
        num_cores=NUM_CORES, num_subcores=NUM_SUBCORES),
    scratch_types=[
        pltpu.VMEM((GL,), jnp.float32),
        pltpu.VMEM((GL,), jnp.float32),
    ],
    compiler_params=pltpu.CompilerParams(
        use_tc_tiling_on_sc=False, needs_layout_passes=False),
)(_body)


def kernel(input):
    return _sparsemax_sc(input.reshape(N * C)).reshape(N, C)


# speedup vs baseline: 10.1797x; 5.2040x over previous
import functools

import jax
import jax.numpy as jnp
from jax import lax
from jax.experimental import pallas as pl
from jax.experimental.pallas import tpu as pltpu
from jax.experimental.pallas import tpu_sc as plsc

N = 8192
C = 2048
L = 16
NUM_CORES = 2
NUM_SUBCORES = 16
NW = NUM_CORES * NUM_SUBCORES
ROWS_PER_W = N // NW
GROUPS = ROWS_PER_W // L
BISECT_ITERS = 20
U = 8
GL = L * C


def _body(x_hbm, out_hbm, buf, buf_t):
    cid = lax.axis_index("c")
    sid = lax.axis_index("s")
    wid = sid * NUM_CORES + cid
    rowbase = lax.iota(jnp.int32, L) * C

    def do_group(g, carry):
        base = (wid * GROUPS + g) * GL
        pltpu.sync_copy(x_hbm.at[pl.ds(base, GL)], buf)

        @plsc.parallel_loop(0, C, step=U,
                            carry=(jnp.full((L,), -1e30, jnp.float32),) * U)
        def trans_loop(j, ms):
            out = []
            for u in range(U):
                v = plsc.load_gather(buf, [rowbase + lax.broadcast(j + u, (L,))])
                buf_t[pl.ds((j + u) * L, L)] = v
                out.append(jnp.maximum(ms[u], v))
            return tuple(out)

        m = functools.reduce(jnp.maximum, trans_loop)

        def bisect(i, lohi):
            lo, hi = lohi
            mid = 0.5 * (lo + hi)

            @plsc.parallel_loop(0, GL, step=U * L,
                                carry=(jnp.zeros((L,), jnp.float32),) * U)
            def acc_loop(o, accs):
                return tuple(
                    accs[u]
                    + jnp.maximum(buf_t[pl.ds(o + u * L, L)] - mid, 0.0)
                    for u in range(U))

            s = functools.reduce(jnp.add, acc_loop)
            ge = s >= 1.0
            return jnp.where(ge, mid, lo), jnp.where(ge, hi, mid)

        lo, hi = lax.fori_loop(0, BISECT_ITERS, bisect, (m - 1.0, m))

        @plsc.parallel_loop(0, GL, step=U * L,
                            carry=(jnp.zeros((L,), jnp.float32),) * (2 * U))
        def ks_loop(o, ks):
            ks = list(ks)
            for u in range(U):
                v = buf_t[pl.ds(o + u * L, L)]
                act = v > lo
                ks[2 * u] = ks[2 * u] + jnp.where(act, 1.0, 0.0)
                ks[2 * u + 1] = ks[2 * u + 1] + jnp.where(act, v, 0.0)
            return tuple(ks)

        k = functools.reduce(jnp.add, ks_loop[0::2])
        s = functools.reduce(jnp.add, ks_loop[1::2])
        tau = (s - 1.0) / k

        @plsc.parallel_loop(0, C, step=U)
        def out_loop(j):
            for u in range(U):
                v = jnp.maximum(buf_t[pl.ds((j + u) * L, L)] - tau, 0.0)
                plsc.store_scatter(
                    buf, [rowbase + lax.broadcast(j + u, (L,))], v)

        pltpu.sync_copy(buf, out_hbm.at[pl.ds(base, GL)])
        return carry

    lax.fori_loop(0, GROUPS, do_group, 0)


_sparsemax_sc = functools.partial(
    pl.kernel,
    out_type=jax.ShapeDtypeStruct((N * C,), jnp.float32),
    mesh=plsc.VectorSubcoreMesh(
        core_axis_name="c", subcore_axis_name="s",
        num_cores=NUM_CORES, num_subcores=NUM_SUBCORES),
    scratch_types=[
        pltpu.VMEM((GL,), jnp.float32),
        pltpu.VMEM((GL,), jnp.float32),
    ],
    compiler_params=pltpu.CompilerParams(
        use_tc_tiling_on_sc=False, needs_layout_passes=False),
)(_body)


def kernel(input):
    return _sparsemax_sc(input.reshape(N * C)).reshape(N, C)


# speedup vs baseline: 12.6689x; 1.2445x over previous
import functools

import jax
import jax.numpy as jnp
from jax import lax
from jax.experimental import pallas as pl
from jax.experimental.pallas import tpu as pltpu
from jax.experimental.pallas import tpu_sc as plsc

N = 8192
C = 2048
L = 16
NUM_CORES = 2
NUM_SUBCORES = 16
NW = NUM_CORES * NUM_SUBCORES
ROWS_PER_W = N // NW
GROUPS = ROWS_PER_W // L
U = 8
GL = L * C
NB = 256
HL = L * NB


def _body(x_hbm, out_hbm, buf, buf_t, hcnt, hsum):
    cid = lax.axis_index("c")
    sid = lax.axis_index("s")
    wid = sid * NUM_CORES + cid
    rowbase = lax.iota(jnp.int32, L) * C
    histbase = lax.iota(jnp.int32, L) * NB
    ones = jnp.ones((L,), jnp.float32)
    zeros = jnp.zeros((L,), jnp.float32)

    def do_group(g, carry):
        base = (wid * GROUPS + g) * GL
        pltpu.sync_copy(x_hbm.at[pl.ds(base, GL)], buf)

        @plsc.parallel_loop(0, C, step=U,
                            carry=(jnp.full((L,), -1e30, jnp.float32),) * U)
        def trans_loop(j, ms):
            out = []
            for u in range(U):
                v = plsc.load_gather(buf, [rowbase + lax.broadcast(j + u, (L,))])
                buf_t[pl.ds((j + u) * L, L)] = v
                out.append(jnp.maximum(ms[u], v))
            return tuple(out)

        m = functools.reduce(jnp.maximum, trans_loop)

        @plsc.parallel_loop(0, NB, step=U)
        def zero_loop(j):
            for u in range(U):
                hcnt[pl.ds((j + u) * L, L)] = zeros
                hsum[pl.ds((j + u) * L, L)] = zeros

        @plsc.parallel_loop(0, GL, step=U * L)
        def hist_loop(o):
            for u in range(U):
                v = buf_t[pl.ds(o + u * L, L)]
                e = (m - v) * 256.0
                msk = e < 256.0
                idx = histbase + e.astype(jnp.int32)
                plsc.addupdate_scatter(hcnt, [idx], ones, mask=msk)
                plsc.addupdate_scatter(hsum, [idx], v, mask=msk)

        def scan_step(j, st):
            Kc, Sc, Kat, Sat, found = st
            idx = histbase + lax.broadcast(j, (L,))
            Kc = Kc + plsc.load_gather(hcnt, [idx])
            Sc = Sc + plsc.load_gather(hsum, [idx])
            jf = lax.broadcast(j + 1, (L,)).astype(jnp.float32)
            tj = m - jf * (1.0 / 256.0)
            cross = jnp.logical_and(Sc - tj * Kc >= 1.0,
                                    jnp.logical_not(found))
            Kat = jnp.where(cross, Kc, Kat)
            Sat = jnp.where(cross, Sc, Sat)
            return Kc, Sc, Kat, Sat, jnp.logical_or(found, cross)

        _, _, Kat, Sat, _ = lax.fori_loop(
            0, NB, scan_step,
            (zeros, zeros, ones, m, jnp.zeros((L,), jnp.bool_)))
        tau = (Sat - 1.0) / Kat

        for _ in range(2):
            t = tau

            @plsc.parallel_loop(0, GL, step=U * L,
                                carry=(zeros,) * (2 * U))
            def ks_loop(o, ks):
                ks = list(ks)
                for u in range(U):
                    v = buf_t[pl.ds(o + u * L, L)]
                    act = v > t
                    ks[2 * u] = ks[2 * u] + jnp.where(act, 1.0, 0.0)
                    ks[2 * u + 1] = ks[2 * u + 1] + jnp.where(act, v, 0.0)
                return tuple(ks)

            k = functools.reduce(jnp.add, ks_loop[0::2])
            s = functools.reduce(jnp.add, ks_loop[1::2])
            tau = (s - 1.0) / k

        @plsc.parallel_loop(0, C, step=U)
        def out_loop(j):
            for u in range(U):
                v = jnp.maximum(buf_t[pl.ds((j + u) * L, L)] - tau, 0.0)
                plsc.store_scatter(
                    buf, [rowbase + lax.broadcast(j + u, (L,))], v)

        pltpu.sync_copy(buf, out_hbm.at[pl.ds(base, GL)])
        return carry

    lax.fori_loop(0, GROUPS, do_group, 0)


_sparsemax_sc = functools.partial(
    pl.kernel,
    out_type=jax.ShapeDtypeStruct((N * C,), jnp.float32),
    mesh=plsc.VectorSubcoreMesh(
        core_axis_name="c", subcore_axis_name="s",
        num_cores=NUM_CORES, num_subcores=NUM_SUBCORES),
    scratch_types=[
        pltpu.VMEM((GL,), jnp.float32),
        pltpu.VMEM((GL,), jnp.float32),
        pltpu.VMEM((HL,), jnp.float32),
        pltpu.VMEM((HL,), jnp.float32),
    ],
    compiler_params=pltpu.CompilerParams(
        use_tc_tiling_on_sc=False, needs_layout_passes=False),
)(_body)


def kernel(input):
    return _sparsemax_sc(input.reshape(N * C)).reshape(N, C)
